# baseline (speedup 1.0000x reference)
import functools

import jax
import jax.numpy as jnp
from jax import lax
from jax.experimental import pallas as pl
from jax.experimental.pallas import tpu as pltpu
from jax.experimental.pallas import tpu_sc as plsc

_ALPHA = 4
_NC = 2
_NS = 16

_NBUF = 6
_RAHEAD = 3

_SC_NBUF = 4
_SC_RAHEAD = 2


def _dma_ring(rd, wr, n, nbuf, rahead):
    for i in range(min(rahead, n)):
        rd(i).start()
    for u in range(n):
        rd(u).wait()
        wr(u).start()
        m = u + rahead
        if m < n:
            if m >= nbuf:
                wr(m - nbuf).wait()
            rd(m).start()
    for u in range(max(0, n - nbuf), n):
        wr(u).wait()


def _tc_copy_body(blk_t, x_hbm, o_hbm, *rest):
    B, T = x_hbm.shape[0], x_hbm.shape[1]
    units_per_b = T // blk_t
    n_units = B * units_per_b
    bufs = rest[:_NBUF]
    rsems = rest[_NBUF:2 * _NBUF]
    wsems = rest[2 * _NBUF:3 * _NBUF]

    def rd(u):
        b, t0 = u // units_per_b, (u % units_per_b) * blk_t
        return pltpu.make_async_copy(
            x_hbm.at[b, pl.ds(t0, blk_t)], bufs[u % _NBUF], rsems[u % _NBUF])

    def wr(u):
        b, t0 = u // units_per_b, (u % units_per_b) * blk_t
        return pltpu.make_async_copy(
            bufs[u % _NBUF], o_hbm.at[b, pl.ds(t0, blk_t)], wsems[u % _NBUF])

    _dma_ring(rd, wr, n_units, _NBUF, _RAHEAD)


def _sc_gather_body(T, S, frames_hbm, out_hbm, *rest):
    c = lax.axis_index("c")
    s = lax.axis_index("s")
    w = c * _NS + s
    b = w // S
    j = w % S
    t = (j * (T - 1)) // (S - 1)

    bufs = rest[:_SC_NBUF]
    rsems = rest[_SC_NBUF:2 * _SC_NBUF]
    wsems = rest[2 * _SC_NBUF:3 * _SC_NBUF]

    C, H = frames_hbm.shape[2], frames_hbm.shape[3]
    hh = H // 2
    n_units = C * 2

    def rd(u):
        ch, h0 = u // 2, (u % 2) * hh
        return pltpu.make_async_copy(
            frames_hbm.at[b, t, ch, pl.ds(h0, hh)],
            bufs[u % _SC_NBUF], rsems[u % _SC_NBUF])

    def wr(u):
        ch, h0 = u // 2, (u % 2) * hh
        return pltpu.make_async_copy(
            bufs[u % _SC_NBUF],
            out_hbm.at[b, j, ch, pl.ds(h0, hh)], wsems[u % _SC_NBUF])

    _dma_ring(rd, wr, n_units, _SC_NBUF, _SC_RAHEAD)


def kernel(frames):
    B, T, C, H, W = frames.shape
    S = T // _ALPHA
    assert B * S == _NC * _NS, "one slow frame per SC vector subcore"
    import numpy as _np
    _expect = _np.linspace(0.0, T - 1, S).astype(_np.int32)
    _got = (_np.arange(S) * (T - 1)) // (S - 1)
    assert _np.array_equal(_expect, _got), (_expect, _got)

    slow = pl.kernel(
        functools.partial(_sc_gather_body, T, S),
        out_type=jax.ShapeDtypeStruct((B, S, C, H, W), jnp.float32),
        mesh=plsc.VectorSubcoreMesh(core_axis_name="c", subcore_axis_name="s"),
        scratch_types=(
            [pltpu.VMEM((H // 2, W), jnp.float32)] * _SC_NBUF
            + [pltpu.SemaphoreType.DMA] * (2 * _SC_NBUF)
        ),
    )(frames)

    blk_t = 8
    fast = pl.pallas_call(
        functools.partial(_tc_copy_body, blk_t),
        in_specs=[pl.BlockSpec(memory_space=pltpu.HBM)],
        out_specs=pl.BlockSpec(memory_space=pltpu.HBM),
        out_shape=jax.ShapeDtypeStruct((B, T, C, H, W), jnp.float32),
        scratch_shapes=(
            [pltpu.VMEM((blk_t, C, H, W), jnp.float32)] * _NBUF
            + [pltpu.SemaphoreType.DMA] * (2 * _NBUF)
        ),
    )(frames)

    return (slow, fast)


# speedup vs baseline: 1.0000x; 1.0000x over previous
import functools

import jax
import jax.numpy as jnp
from jax import lax
from jax.experimental import pallas as pl
from jax.experimental.pallas import tpu as pltpu
from jax.experimental.pallas import tpu_sc as plsc

_ALPHA = 4
_NC = 2
_NS = 16

_NBUF = 4
_RAHEAD = 2

_SC_NBUF = 4
_SC_RAHEAD = 2


def _dma_ring(rd, wr, n, nbuf, rahead):
    for i in range(min(rahead, n)):
        rd(i).start()
    for u in range(n):
        rd(u).wait()
        wr(u).start()
        m = u + rahead
        if m < n:
            if m >= nbuf:
                wr(m - nbuf).wait()
            rd(m).start()
    for u in range(max(0, n - nbuf), n):
        wr(u).wait()


def _tc_copy_body(blk_t, x_hbm, o_hbm, *rest):
    B, T = x_hbm.shape[0], x_hbm.shape[1]
    units_per_b = T // blk_t
    n_units = B * units_per_b
    bufs = rest[:_NBUF]
    rsems = rest[_NBUF:2 * _NBUF]
    wsems = rest[2 * _NBUF:3 * _NBUF]

    def rd(u):
        b, t0 = u // units_per_b, (u % units_per_b) * blk_t
        return pltpu.make_async_copy(
            x_hbm.at[b, pl.ds(t0, blk_t)], bufs[u % _NBUF], rsems[u % _NBUF])

    def wr(u):
        b, t0 = u // units_per_b, (u % units_per_b) * blk_t
        return pltpu.make_async_copy(
            bufs[u % _NBUF], o_hbm.at[b, pl.ds(t0, blk_t)], wsems[u % _NBUF])

    _dma_ring(rd, wr, n_units, _NBUF, _RAHEAD)


def _sc_gather_body(T, S, frames_hbm, out_hbm, *rest):
    c = lax.axis_index("c")
    s = lax.axis_index("s")
    w = c * _NS + s
    b = w // S
    j = w % S
    t = (j * (T - 1)) // (S - 1)

    bufs = rest[:_SC_NBUF]
    rsems = rest[_SC_NBUF:2 * _SC_NBUF]
    wsems = rest[2 * _SC_NBUF:3 * _SC_NBUF]

    C, H = frames_hbm.shape[2], frames_hbm.shape[3]
    hh = H // 2
    n_units = C * 2

    def rd(u):
        ch, h0 = u // 2, (u % 2) * hh
        return pltpu.make_async_copy(
            frames_hbm.at[b, t, ch, pl.ds(h0, hh)],
            bufs[u % _SC_NBUF], rsems[u % _SC_NBUF])

    def wr(u):
        ch, h0 = u // 2, (u % 2) * hh
        return pltpu.make_async_copy(
            bufs[u % _SC_NBUF],
            out_hbm.at[b, j, ch, pl.ds(h0, hh)], wsems[u % _SC_NBUF])

    _dma_ring(rd, wr, n_units, _SC_NBUF, _SC_RAHEAD)


def kernel(frames):
    B, T, C, H, W = frames.shape
    S = T // _ALPHA
    assert B * S == _NC * _NS, "one slow frame per SC vector subcore"
    import numpy as _np
    _expect = _np.linspace(0.0, T - 1, S).astype(_np.int32)
    _got = (_np.arange(S) * (T - 1)) // (S - 1)
    assert _np.array_equal(_expect, _got), (_expect, _got)

    slow = pl.kernel(
        functools.partial(_sc_gather_body, T, S),
        out_type=jax.ShapeDtypeStruct((B, S, C, H, W), jnp.float32),
        mesh=plsc.VectorSubcoreMesh(core_axis_name="c", subcore_axis_name="s"),
        scratch_types=(
            [pltpu.VMEM((H // 2, W), jnp.float32)] * _SC_NBUF
            + [pltpu.SemaphoreType.DMA] * (2 * _SC_NBUF)
        ),
    )(frames)

    blk_t = 16
    fast = pl.pallas_call(
        functools.partial(_tc_copy_body, blk_t),
        in_specs=[pl.BlockSpec(memory_space=pltpu.HBM)],
        out_specs=pl.BlockSpec(memory_space=pltpu.HBM),
        out_shape=jax.ShapeDtypeStruct((B, T, C, H, W), jnp.float32),
        scratch_shapes=(
            [pltpu.VMEM((blk_t, C, H, W), jnp.float32)] * _NBUF
            + [pltpu.SemaphoreType.DMA] * (2 * _NBUF)
        ),
    )(frames)

    return (slow, fast)
